# baseline (speedup 1.0000x reference)
import functools

import jax
import jax.numpy as jnp
from jax import lax
from jax.experimental import pallas as pl
from jax.experimental.pallas import tpu as pltpu
from jax.experimental.pallas import tpu_sc as plsc


def _lookup_kernel(B, D, V, NW, b_per_w, C):
    mesh = plsc.VectorSubcoreMesh(core_axis_name="c", subcore_axis_name="s")
    n_chunks = b_per_w // C
    n_pieces = D // 16
    GR = 16

    @functools.partial(
        pl.kernel,
        mesh=mesh,
        out_type=jax.ShapeDtypeStruct((B * D,), jnp.float32),
        scratch_types=[
            pltpu.VMEM((b_per_w,), jnp.int32),
            pltpu.VMEM((V * D,), jnp.float32),
            pltpu.VMEM((2 * C * D,), jnp.float32),
            pltpu.SemaphoreType.DMA,
            pltpu.SemaphoreType.DMA,
        ],
    )
    def k(x_hbm, table_hbm, out_hbm, idx_v, table_f, rows_f, s0, s1):
        wid = lax.axis_index("s") * 2 + lax.axis_index("c")
        base = wid * b_per_w
        pltpu.sync_copy(x_hbm.at[pl.ds(base, b_per_w)], idx_v)
        pltpu.sync_copy(table_hbm, table_f)

        ssems = (s0, s1)

        def store(c, b):
            return pltpu.make_async_copy(
                rows_f.at[pl.ds(b * C * D, C * D)],
                out_hbm.at[pl.ds((base + c * C) * D, C * D)],
                ssems[b],
            )

        def assemble(c, b):
            def grp(g, _):
                iv = idx_v[pl.ds(c * C + g * GR, GR)]
                db0 = b * C * D + g * (GR * D)
                for r in range(GR):
                    tb = iv[r] * D
                    db = db0 + r * D
                    for p in range(n_pieces):
                        rows_f[pl.ds(db + p * 16, 16)] = table_f[
                            pl.ds(tb + p * 16, 16)
                        ]
                return _

            lax.fori_loop(0, C // GR, grp, None)

        assemble(0, 0)
        store(0, 0).start()

        def body(c, _):
            for b in range(2):
                cc = c + b
                nxt = 1 - b
                @pl.when(cc >= 1)
                def _():
                    store(cc - 1, nxt).wait()

                @pl.when(cc + 1 < n_chunks)
                def _():
                    assemble(cc + 1, nxt)
                    store(cc + 1, nxt).start()

            return _

        lax.fori_loop(0, n_chunks // 2, lambda c, u: body(c * 2, u), None)
        store(n_chunks - 1, (n_chunks - 1) % 2).wait()

    return k


def kernel(x, table):
    S, J = x.shape
    V, D = table.shape
    B = S * J
    NW = 32
    b_per_w = B // NW
    C = 64
    xf = x.reshape(B).astype(jnp.int32)
    out = _lookup_kernel(B, D, V, NW, b_per_w, C)(xf, table.reshape(V * D))
    return out.reshape(S, J, D)


# speedup vs baseline: 1.7387x; 1.7387x over previous
import functools

import jax
import jax.numpy as jnp
from jax import lax
from jax.experimental import pallas as pl
from jax.experimental.pallas import tpu as pltpu
from jax.experimental.pallas import tpu_sc as plsc


def _gather_kernel(BP, D2, NW, p_per_w, C):
    mesh = plsc.VectorSubcoreMesh(core_axis_name="c", subcore_axis_name="s")
    n_chunks = p_per_w // C

    @functools.partial(
        pl.kernel,
        mesh=mesh,
        out_type=jax.ShapeDtypeStruct((BP, D2), jnp.float32),
        scratch_types=[
            pltpu.VMEM((p_per_w,), jnp.int32),
            pltpu.VMEM((2, C, D2), jnp.float32),
            pltpu.SemaphoreType.DMA,
            pltpu.SemaphoreType.DMA,
            pltpu.SemaphoreType.DMA,
            pltpu.SemaphoreType.DMA,
        ],
    )
    def k(xp_hbm, pt_hbm, out_hbm, idx_v, rows_v, g0, g1, s0, s1):
        wid = lax.axis_index("s") * 2 + lax.axis_index("c")
        base = wid * p_per_w
        pltpu.sync_copy(xp_hbm.at[pl.ds(base, p_per_w)], idx_v)

        gsems = (g0, g1)
        ssems = (s0, s1)

        def gather(c, b):
            return pltpu.make_async_copy(
                pt_hbm.at[idx_v.at[pl.ds(c * C, C)]], rows_v.at[b], gsems[b]
            )

        def store(c, b):
            return pltpu.make_async_copy(
                rows_v.at[b], out_hbm.at[pl.ds(base + c * C, C)], ssems[b]
            )

        gather(0, 0).start()

        def body(c, _):
            for b in range(2):
                cc = c + b
                gather(cc, b).wait()
                store(cc, b).start()
                @pl.when(cc >= 1)
                def _():
                    store(cc - 1, 1 - b).wait()

                @pl.when(cc + 1 < n_chunks)
                def _():
                    gather(cc + 1, 1 - b).start()

            return _

        lax.fori_loop(0, n_chunks // 2, lambda c, u: body(c * 2, u), None)
        store(n_chunks - 1, (n_chunks - 1) % 2).wait()

    return k


def kernel(x, table):
    S, J = x.shape
    V, D = table.shape
    B = S * J
    BP = B // 2
    D2 = 2 * D
    NW = 32
    p_per_w = BP // NW
    C = 32
    pt = jnp.concatenate(
        [jnp.repeat(table, V, axis=0), jnp.tile(table, (V, 1))], axis=1
    )
    xf = x.reshape(BP, 2).astype(jnp.int32)
    xpi = xf[:, 0] * V + xf[:, 1]
    out = _gather_kernel(BP, D2, NW, p_per_w, C)(xpi, pt)
    return out.reshape(S, J, D)


# speedup vs baseline: 1.8106x; 1.0413x over previous
import functools

import jax
import jax.numpy as jnp
from jax import lax
from jax.experimental import pallas as pl
from jax.experimental.pallas import tpu as pltpu
from jax.experimental.pallas import tpu_sc as plsc


def _gather_kernel(BP, D2, NW, p_per_w, C):
    mesh = plsc.VectorSubcoreMesh(core_axis_name="c", subcore_axis_name="s")
    n_chunks = p_per_w // C

    @functools.partial(
        pl.kernel,
        mesh=mesh,
        out_type=jax.ShapeDtypeStruct((BP, D2), jnp.float32),
        scratch_types=[
            pltpu.VMEM((p_per_w,), jnp.int32),
            pltpu.VMEM((2, C, D2), jnp.float32),
            pltpu.SemaphoreType.DMA,
            pltpu.SemaphoreType.DMA,
            pltpu.SemaphoreType.DMA,
            pltpu.SemaphoreType.DMA,
        ],
    )
    def k(xp_hbm, pt_hbm, out_hbm, idx_v, rows_v, g0, g1, s0, s1):
        wid = lax.axis_index("s") * 2 + lax.axis_index("c")
        base = wid * p_per_w
        pltpu.sync_copy(xp_hbm.at[pl.ds(base, p_per_w)], idx_v)

        gsems = (g0, g1)
        ssems = (s0, s1)

        def gather(c, b):
            return pltpu.make_async_copy(
                pt_hbm.at[idx_v.at[pl.ds(c * C, C)]], rows_v.at[b], gsems[b]
            )

        def store(c, b):
            return pltpu.make_async_copy(
                rows_v.at[b], out_hbm.at[pl.ds(base + c * C, C)], ssems[b]
            )

        gather(0, 0).start()

        def body(c, _):
            for b in range(2):
                cc = c + b
                gather(cc, b).wait()
                store(cc, b).start()
                @pl.when(cc >= 1)
                def _():
                    store(cc - 1, 1 - b).wait()

                @pl.when(cc + 1 < n_chunks)
                def _():
                    gather(cc + 1, 1 - b).start()

            return _

        lax.fori_loop(0, n_chunks // 2, lambda c, u: body(c * 2, u), None)
        store(n_chunks - 1, (n_chunks - 1) % 2).wait()

    return k


def kernel(x, table):
    S, J = x.shape
    V, D = table.shape
    B = S * J
    BP = B // 2
    D2 = 2 * D
    NW = 32
    p_per_w = BP // NW
    C = 32
    pt = jnp.concatenate(
        [jnp.repeat(table, V, axis=0), jnp.tile(table, (V, 1))], axis=1
    )
    NR = 8
    pt = jnp.tile(pt, (NR, 1))
    xf = x.reshape(BP, 2).astype(jnp.int32)
    xpi = xf[:, 0] * V + xf[:, 1]
    xpi = xpi + (V * V) * (jnp.arange(BP, dtype=jnp.int32) // (BP // NR))
    out = _gather_kernel(BP, D2, NW, p_per_w, C)(xpi, pt)
    return out.reshape(S, J, D)


# speedup vs baseline: 3.3695x; 1.8610x over previous
import functools

import jax
import jax.numpy as jnp
from jax import lax
from jax.experimental import pallas as pl
from jax.experimental.pallas import tpu as pltpu
from jax.experimental.pallas import tpu_sc as plsc


def _gather_kernel(B, D, V, NW, b_per_w, C):
    mesh = plsc.VectorSubcoreMesh(core_axis_name="c", subcore_axis_name="s")
    n_chunks = b_per_w // C

    @functools.partial(
        pl.kernel,
        mesh=mesh,
        out_type=jax.ShapeDtypeStruct((B, D), jnp.float32),
        scratch_types=[
            pltpu.VMEM((b_per_w,), jnp.int32),
            pltpu.VMEM((2, C, D), jnp.float32),
            pltpu.SemaphoreType.DMA,
            pltpu.SemaphoreType.DMA,
            pltpu.SemaphoreType.DMA,
            pltpu.SemaphoreType.DMA,
        ],
    )
    def k(x_hbm, table_hbm, out_hbm, idx_v, rows_v, g0, g1, s0, s1):
        wid = lax.axis_index("s") * 2 + lax.axis_index("c")
        base = wid * b_per_w
        pltpu.sync_copy(x_hbm.at[pl.ds(base, b_per_w)], idx_v)

        gsems = (g0, g1)
        ssems = (s0, s1)

        def gather(c, b):
            return pltpu.make_async_copy(
                table_hbm.at[idx_v.at[pl.ds(c * C, C)]], rows_v.at[b], gsems[b]
            )

        def store(c, b):
            return pltpu.make_async_copy(
                rows_v.at[b], out_hbm.at[pl.ds(base + c * C, C)], ssems[b]
            )

        gather(0, 0).start()

        def body(c, _):
            for b in range(2):
                cc = c + b
                gather(cc, b).wait()
                store(cc, b).start()
                @pl.when(cc >= 1)
                def _():
                    store(cc - 1, 1 - b).wait()

                @pl.when(cc + 1 < n_chunks)
                def _():
                    gather(cc + 1, 1 - b).start()

            return _

        lax.fori_loop(0, n_chunks // 2, lambda c, u: body(c * 2, u), None)
        store(n_chunks - 1, (n_chunks - 1) % 2).wait()

    return k


def kernel(x, table):
    S, J = x.shape
    V, D = table.shape
    B = S * J
    NW = 32
    b_per_w = B // NW
    C = 64
    PADR = 64
    table_rep = jnp.concatenate(
        [
            jnp.broadcast_to(table, (NW, V, D)),
            jnp.zeros((NW, PADR - V, D), jnp.float32),
        ],
        axis=1,
    ).reshape(NW * PADR, D)
    xf = x.reshape(B).astype(jnp.int32)
    xf = xf + PADR * (jnp.arange(B, dtype=jnp.int32) // b_per_w)
    out = _gather_kernel(B, D, V, NW, b_per_w, C)(xf, table_rep)
    return out.reshape(S, J, D)
